# baseline (speedup 1.0000x reference)
import jax
import jax.numpy as jnp
from jax import lax
from jax.experimental import pallas as pl
from jax.experimental.pallas import tpu as pltpu
from jax.experimental.pallas import tpu_sc as plsc

D = 64
SCALE = 8.0
LANES = 16
IDXROW = 128
CHUNK = 256
NBUF = 2


def kernel(x, lut):
    b0, hist = x.shape
    nw = 32
    bw = b0 // nw
    k = CHUNK // IDXROW
    cw = bw // CHUNK
    nchunks = hist * cw
    xt3 = jnp.transpose(x).reshape(hist, b0 // IDXROW, IDXROW)

    mesh = plsc.VectorSubcoreMesh(core_axis_name="c", subcore_axis_name="s")

    @pl.kernel(
        mesh=mesh,
        compiler_params=pltpu.CompilerParams(
            use_tc_tiling_on_sc=False, needs_layout_passes=False
        ),
        out_type=jax.ShapeDtypeStruct((hist, D, b0), jnp.float32),
        scratch_types=[
            [pltpu.VMEM((k, IDXROW), jnp.int32) for _ in range(NBUF)],
            [pltpu.VMEM((CHUNK, D), jnp.float32) for _ in range(NBUF)],
            [pltpu.VMEM((D, CHUNK), jnp.float32) for _ in range(NBUF)],
            [pltpu.SemaphoreType.DMA for _ in range(NBUF)],
            [pltpu.SemaphoreType.DMA for _ in range(NBUF)],
        ],
    )
    def emb(idx_hbm, tab_hbm, out_hbm, idx_v, rows_v, tr_v, gsem, osem):
        wid = lax.axis_index("s") * 2 + lax.axis_index("c")
        jrow0 = wid * (bw // IDXROW)
        bbase = wid * bw
        iota = lax.iota(jnp.int32, LANES)
        skew = [(iota + s) & (LANES - 1) for s in range(LANES)]

        def stage_and_gather(c, buf):
            h = c // cw
            par = c % cw
            pltpu.sync_copy(
                idx_hbm.at[h, pl.ds(jrow0 + par * k, k)], idx_v[buf]
            )
            for j in range(k):
                pltpu.async_copy(
                    tab_hbm.at[idx_v[buf].at[j]],
                    rows_v[buf].at[pl.ds(j * IDXROW, IDXROW)],
                    gsem[buf],
                )

        def drain_rows(sem, buf):
            pltpu.make_async_copy(
                tab_hbm.at[pl.ds(0, CHUNK)], rows_v[buf], sem
            ).wait()

        def drain_out(sem, buf):
            pltpu.make_async_copy(
                out_hbm.at[0, :, pl.ds(0, CHUNK)], tr_v[buf], sem
            ).wait()

        stage_and_gather(0, 0)

        @pl.loop(0, nchunks, step=NBUF)
        def chunk_body(c0):
            for phase in range(NBUF):
                c = c0 + phase
                cur = phase
                nxt = (phase + 1) % NBUF

                @pl.when(c + 1 < nchunks)
                def _prefetch():
                    stage_and_gather(c + 1, nxt)

                drain_rows(gsem[cur], cur)

                @pl.when(c >= NBUF)
                def _wait_out():
                    drain_out(osem[cur], cur)

                @plsc.parallel_loop(0, D // LANES, unroll=1)
                def trans_dg(dg):
                    d0 = dg * LANES
                    for bg in range(CHUNK // LANES):
                        bvec = iota + (bg * LANES)
                        for s in range(LANES):
                            dvec = skew[s] + d0
                            vals = plsc.load_gather(
                                rows_v[cur], [bvec, dvec]
                            )
                            plsc.store_scatter(
                                tr_v[cur], [dvec, bvec], vals * SCALE
                            )

                h = c // cw
                par = c % cw
                pltpu.async_copy(
                    tr_v[cur],
                    out_hbm.at[h, :, pl.ds(bbase + par * CHUNK, CHUNK)],
                    osem[cur],
                )

        for buf in range(NBUF):
            drain_out(osem[buf], buf)

    out3 = emb(xt3, lut)
    return jnp.transpose(out3, (2, 0, 1))


# speedup vs baseline: 1.5444x; 1.5444x over previous
import jax
import jax.numpy as jnp
from jax import lax
from jax.experimental import pallas as pl
from jax.experimental.pallas import tpu as pltpu
from jax.experimental.pallas import tpu_sc as plsc

D = 64
SCALE = 8.0
LANES = 16
IDXROW = 128
CHUNK = 256
NBUF = 2


def kernel(x, lut):
    b0, hist = x.shape
    b = b0 * hist
    nw = 32
    b_per_w = b // nw
    k = CHUNK // IDXROW
    nchunks = b_per_w // CHUNK
    x2d = x.reshape(b // IDXROW, IDXROW).astype(jnp.int32)
    lut_p = jnp.pad(lut, ((0, 0), (0, 2 * D - lut.shape[1])))

    mesh = plsc.VectorSubcoreMesh(core_axis_name="c", subcore_axis_name="s")

    @pl.kernel(
        mesh=mesh,
        compiler_params=pltpu.CompilerParams(
            use_tc_tiling_on_sc=False, needs_layout_passes=False
        ),
        out_type=jax.ShapeDtypeStruct((b, D), jnp.float32),
        scratch_types=[
            [pltpu.VMEM((k, IDXROW), jnp.int32) for _ in range(NBUF)],
            [pltpu.VMEM((CHUNK, 2 * D), jnp.float32) for _ in range(NBUF)],
            [pltpu.VMEM((CHUNK, D), jnp.float32) for _ in range(NBUF)],
            [pltpu.SemaphoreType.DMA for _ in range(NBUF)],
            [pltpu.SemaphoreType.DMA for _ in range(NBUF)],
        ],
    )
    def emb(idx_hbm, tab_hbm, out_hbm, idx_v, rows_v, out_v, gsem, osem):
        wid = lax.axis_index("s") * 2 + lax.axis_index("c")
        irow0 = wid * (b_per_w // IDXROW)
        obase0 = wid * b_per_w
        out2 = out_hbm

        def stage_and_gather(c, buf):
            pltpu.sync_copy(idx_hbm.at[pl.ds(irow0 + c * k, k)], idx_v[buf])
            for j in range(k):
                pltpu.async_copy(
                    tab_hbm.at[idx_v[buf].at[j]],
                    rows_v[buf].at[pl.ds(j * IDXROW, IDXROW)],
                    gsem[buf],
                )

        def drain_rows(sem, buf):
            pltpu.make_async_copy(
                tab_hbm.at[pl.ds(0, CHUNK)], rows_v[buf], sem
            ).wait()

        def drain_out(sem, buf):
            pltpu.make_async_copy(
                out2.at[pl.ds(0, CHUNK)], out_v[buf], sem
            ).wait()

        stage_and_gather(0, 0)

        @pl.loop(0, nchunks, step=NBUF)
        def chunk_body(c0):
            for phase in range(NBUF):
                c = c0 + phase
                cur = phase
                nxt = (phase + 1) % NBUF

                @pl.when(c + 1 < nchunks)
                def _prefetch():
                    stage_and_gather(c + 1, nxt)

                drain_rows(gsem[cur], cur)

                @pl.when(c >= NBUF)
                def _wait_out():
                    drain_out(osem[cur], cur)

                @plsc.parallel_loop(0, CHUNK, unroll=4)
                def scale_row(i):
                    for g in range(D // LANES):
                        sl = pl.ds(g * LANES, LANES)
                        out_v[cur][i, sl] = rows_v[cur][i, sl] * SCALE

                pltpu.async_copy(
                    out_v[cur],
                    out2.at[pl.ds(obase0 + c * CHUNK, CHUNK)],
                    osem[cur],
                )

        for buf in range(NBUF):
            drain_out(osem[buf], buf)

    return emb(x2d, lut_p).reshape(b0, hist, D)


# speedup vs baseline: 1.6358x; 1.0592x over previous
import jax
import jax.numpy as jnp
from jax import lax
from jax.experimental import pallas as pl
from jax.experimental.pallas import tpu as pltpu
from jax.experimental.pallas import tpu_sc as plsc

D = 64
SCALE = 8.0
LANES = 16
CHUNKB = 8
NBUF = 2


def kernel(x, lut):
    b0, hist = x.shape
    nw = 32
    bw = b0 // nw
    rows = CHUNKB * hist
    nchunks = bw // CHUNKB
    x2 = (x.astype(jnp.int32) << 1)
    lut_p = jnp.pad(lut, ((0, 0), (0, 2 * D - lut.shape[1])))
    lut2 = lut_p.reshape(2 * lut.shape[0], D)

    mesh = plsc.VectorSubcoreMesh(core_axis_name="c", subcore_axis_name="s")

    @pl.kernel(
        mesh=mesh,
        compiler_params=pltpu.CompilerParams(
            use_tc_tiling_on_sc=False, needs_layout_passes=False
        ),
        out_type=jax.ShapeDtypeStruct((b0, hist, D), jnp.float32),
        scratch_types=[
            [pltpu.VMEM((CHUNKB, hist), jnp.int32) for _ in range(NBUF)],
            [pltpu.VMEM((rows, D), jnp.float32) for _ in range(NBUF)],
            [pltpu.VMEM((CHUNKB, hist, D), jnp.float32) for _ in range(NBUF)],
            [pltpu.SemaphoreType.DMA for _ in range(NBUF)],
            [pltpu.SemaphoreType.DMA for _ in range(NBUF)],
        ],
    )
    def emb(idx_hbm, tab_hbm, out_hbm, idx_v, rows_v, out_v, gsem, osem):
        wid = lax.axis_index("s") * 2 + lax.axis_index("c")
        bbase = wid * bw

        def stage_and_gather(c, buf):
            pltpu.sync_copy(
                idx_hbm.at[pl.ds(bbase + c * CHUNKB, CHUNKB)], idx_v[buf]
            )
            for j in range(CHUNKB):
                pltpu.async_copy(
                    tab_hbm.at[idx_v[buf].at[j]],
                    rows_v[buf].at[pl.ds(j * hist, hist)],
                    gsem[buf],
                )

        def drain_rows(sem, buf):
            pltpu.make_async_copy(
                tab_hbm.at[pl.ds(0, rows)], rows_v[buf], sem
            ).wait()

        def drain_out(sem, buf):
            pltpu.make_async_copy(
                out_hbm.at[pl.ds(0, CHUNKB)], out_v[buf], sem
            ).wait()

        stage_and_gather(0, 0)

        @pl.loop(0, nchunks, step=NBUF)
        def chunk_body(c0):
            for phase in range(NBUF):
                c = c0 + phase
                cur = phase
                nxt = (phase + 1) % NBUF

                @pl.when(c + 1 < nchunks)
                def _prefetch():
                    stage_and_gather(c + 1, nxt)

                drain_rows(gsem[cur], cur)

                @pl.when(c >= NBUF)
                def _wait_out():
                    drain_out(osem[cur], cur)

                for j in range(CHUNKB):
                    @plsc.parallel_loop(0, hist, unroll=2)
                    def scale_row(i):
                        for g in range(D // LANES):
                            sl = pl.ds(g * LANES, LANES)
                            out_v[cur][j, i, sl] = (
                                rows_v[cur][j * hist + i, sl] * SCALE
                            )

                pltpu.async_copy(
                    out_v[cur],
                    out_hbm.at[pl.ds(bbase + c * CHUNKB, CHUNKB)],
                    osem[cur],
                )

        for buf in range(NBUF):
            drain_out(osem[buf], buf)

    return emb(x2, lut2)


# speedup vs baseline: 1.6361x; 1.0001x over previous
import jax
import jax.numpy as jnp
from jax import lax
from jax.experimental import pallas as pl
from jax.experimental.pallas import tpu as pltpu
from jax.experimental.pallas import tpu_sc as plsc

D = 64
SCALE = 8.0
LANES = 16
CHUNKB = 8
NBUF = 2


def kernel(x, lut):
    b0, hist = x.shape
    nw = 32
    bw = b0 // nw
    rows = CHUNKB * hist
    nchunks = bw // CHUNKB
    x2 = (x.astype(jnp.int32) << 1)
    lut_p = jnp.pad(lut, ((0, 0), (0, 2 * D - lut.shape[1])))
    lut2 = lut_p.reshape(2 * lut.shape[0], D)

    mesh = plsc.VectorSubcoreMesh(core_axis_name="c", subcore_axis_name="s")

    @pl.kernel(
        mesh=mesh,
        compiler_params=pltpu.CompilerParams(
            use_tc_tiling_on_sc=False, needs_layout_passes=False
        ),
        out_type=jax.ShapeDtypeStruct((b0 * hist * D // 128, 128), jnp.float32),
        scratch_types=[
            [pltpu.VMEM((CHUNKB, hist), jnp.int32) for _ in range(NBUF)],
            [pltpu.VMEM((rows, D), jnp.float32) for _ in range(NBUF)],
            [pltpu.VMEM((CHUNKB * hist * D // 128, 128), jnp.float32)
             for _ in range(NBUF)],
            [pltpu.SemaphoreType.DMA for _ in range(NBUF)],
            [pltpu.SemaphoreType.DMA for _ in range(NBUF)],
        ],
    )
    def emb(idx_hbm, tab_hbm, out_hbm, idx_v, rows_v, out_v, gsem, osem):
        wid = lax.axis_index("s") * 2 + lax.axis_index("c")
        bbase = wid * bw

        def stage_and_gather(c, buf):
            pltpu.sync_copy(
                idx_hbm.at[pl.ds(bbase + c * CHUNKB, CHUNKB)], idx_v[buf]
            )
            for j in range(CHUNKB):
                pltpu.async_copy(
                    tab_hbm.at[idx_v[buf].at[j]],
                    rows_v[buf].at[pl.ds(j * hist, hist)],
                    gsem[buf],
                )

        def drain_rows(sem, buf):
            pltpu.make_async_copy(
                tab_hbm.at[pl.ds(0, rows)], rows_v[buf], sem
            ).wait()

        orows = CHUNKB * hist * D // 128

        def drain_out(sem, buf):
            pltpu.make_async_copy(
                out_hbm.at[pl.ds(0, orows)], out_v[buf], sem
            ).wait()

        stage_and_gather(0, 0)

        @pl.loop(0, nchunks, step=NBUF)
        def chunk_body(c0):
            for phase in range(NBUF):
                c = c0 + phase
                cur = phase
                nxt = (phase + 1) % NBUF

                @pl.when(c + 1 < nchunks)
                def _prefetch():
                    stage_and_gather(c + 1, nxt)

                drain_rows(gsem[cur], cur)

                @pl.when(c >= NBUF)
                def _wait_out():
                    drain_out(osem[cur], cur)

                @plsc.parallel_loop(0, rows // 2, unroll=2)
                def scale_pair(i):
                    for half in range(2):
                        for g in range(D // LANES):
                            sl = pl.ds(half * D + g * LANES, LANES)
                            rsl = pl.ds(g * LANES, LANES)
                            out_v[cur][i, sl] = (
                                rows_v[cur][2 * i + half, rsl] * SCALE
                            )

                pltpu.async_copy(
                    out_v[cur],
                    out_hbm.at[pl.ds((bbase + c * CHUNKB) * hist * D // 128,
                                     orows)],
                    osem[cur],
                )

        for buf in range(NBUF):
            drain_out(osem[buf], buf)

    return emb(x2, lut2).reshape(b0, hist, D)


# speedup vs baseline: 2.2408x; 1.3696x over previous
import jax
import jax.numpy as jnp
from jax import lax
from jax.experimental import pallas as pl
from jax.experimental.pallas import tpu as pltpu
from jax.experimental.pallas import tpu_sc as plsc

D = 64
SCALE = 8.0
LANES = 16
CHUNKB = 8
NBUF = 2


def kernel(x, lut):
    b0, hist = x.shape
    nw = 32
    bw = b0 // nw
    rows = CHUNKB * hist
    nchunks = bw // CHUNKB
    x2 = (x.astype(jnp.int32) << 1)
    lut_p = jnp.pad(lut, ((0, 0), (0, 2 * D - lut.shape[1])))
    lut2 = lut_p.reshape(2 * lut.shape[0], D)

    mesh = plsc.VectorSubcoreMesh(core_axis_name="c", subcore_axis_name="s")

    @pl.kernel(
        mesh=mesh,
        compiler_params=pltpu.CompilerParams(
            use_tc_tiling_on_sc=False, needs_layout_passes=False
        ),
        out_type=jax.ShapeDtypeStruct((b0, 56, 128), jnp.float32),
        scratch_types=[
            [pltpu.VMEM((CHUNKB, hist), jnp.int32) for _ in range(NBUF)],
            [pltpu.VMEM((rows, D), jnp.float32) for _ in range(NBUF)],
            [pltpu.VMEM((CHUNKB, hist, D), jnp.float32) for _ in range(NBUF)],
            [pltpu.SemaphoreType.DMA for _ in range(NBUF)],
            [pltpu.SemaphoreType.DMA for _ in range(NBUF)],
        ],
    )
    def emb(idx_hbm, tab_hbm, out_hbm, idx_v, rows_v, out_v, gsem, osem):
        wid = lax.axis_index("s") * 2 + lax.axis_index("c")
        bbase = wid * bw

        def stage_and_gather(c, buf):
            pltpu.sync_copy(
                idx_hbm.at[pl.ds(bbase + c * CHUNKB, CHUNKB)], idx_v[buf]
            )
            for j in range(CHUNKB):
                pltpu.async_copy(
                    tab_hbm.at[idx_v[buf].at[j]],
                    rows_v[buf].at[pl.ds(j * hist, hist)],
                    gsem[buf],
                )

        def drain_rows(sem, buf):
            pltpu.make_async_copy(
                tab_hbm.at[pl.ds(0, rows)], rows_v[buf], sem
            ).wait()

        def drain_out(sem, buf):
            pltpu.make_async_copy(
                out_hbm.at[pl.ds(0, CHUNKB), pl.ds(0, hist), pl.ds(0, D)],
                out_v[buf],
                sem,
            ).wait()

        stage_and_gather(0, 0)

        @pl.loop(0, nchunks, step=NBUF)
        def chunk_body(c0):
            for phase in range(NBUF):
                c = c0 + phase
                cur = phase
                nxt = (phase + 1) % NBUF

                @pl.when(c + 1 < nchunks)
                def _prefetch():
                    stage_and_gather(c + 1, nxt)

                drain_rows(gsem[cur], cur)

                @pl.when(c >= NBUF)
                def _wait_out():
                    drain_out(osem[cur], cur)

                for j in range(CHUNKB):
                    @plsc.parallel_loop(0, hist, unroll=2)
                    def scale_row(i):
                        for g in range(D // LANES):
                            sl = pl.ds(g * LANES, LANES)
                            out_v[cur][j, i, sl] = (
                                rows_v[cur][j * hist + i, sl] * SCALE
                            )

                pltpu.async_copy(
                    out_v[cur],
                    out_hbm.at[pl.ds(bbase + c * CHUNKB, CHUNKB),
                               pl.ds(0, hist), pl.ds(0, D)],
                    osem[cur],
                )

        for buf in range(NBUF):
            drain_out(osem[buf], buf)

    return emb(x2, lut2)[:, :hist, :D]
